# baseline (speedup 1.0000x reference)
import functools

import jax
import jax.numpy as jnp
from jax import lax
from jax.experimental import pallas as pl
from jax.experimental.pallas import tpu as pltpu
from jax.experimental.pallas import tpu_sc as plsc

B = 8192
L = 64
M = 16
NUP = 65
A = 4
PLANE = NUP * NUP
PLANE_P = 4288
ROWS_PER_AL = PLANE_P * M // 128
TAB_ROWS = A * L * ROWS_PER_AL
EPS_ROWS = A * M * L * PLANE // 128

NC = 2
NS = 16
NW = NC * NS
SAMPLES_PER_W = B // NW
ROWS_PER_BATCH = 128
N_BATCH = SAMPLES_PER_W * L // ROWS_PER_BATCH
NBUF = 4
GROUP = 8

AL_PER_W = (A * L) // NW
SLAB_ROWS = 48

_CHUNKS = ((0, 128), (1024, 128), (2048, 128), (3072, 128), (4096, 24))

_SC_PARAMS = pltpu.CompilerParams(
    needs_layout_passes=False, use_tc_tiling_on_sc=True)


def _mesh():
    return plsc.VectorSubcoreMesh(
        core_axis_name="c", subcore_axis_name="s", num_cores=NC, num_subcores=NS)


def _idx_body(x_ref, g_ref, k_ref):
    x = x_ref[...]
    up = (x & 1).astype(jnp.float32)
    dn = ((x >> 1) & 1).astype(jnp.float32)
    i = lax.broadcasted_iota(jnp.int32, (128, 128), 0)
    j = lax.broadcasted_iota(jnp.int32, (128, 128), 1)
    tri = ((i < j) & ((i // L) == (j // L))).astype(jnp.float32)
    n_up = jnp.dot(up, tri, preferred_element_type=jnp.float32).astype(jnp.int32)
    n_dn = jnp.dot(dn, tri, preferred_element_type=jnp.float32).astype(jnp.int32)
    site = lax.broadcasted_iota(jnp.int32, x.shape, 1) & (L - 1)
    key = (x * L + site) * PLANE_P + n_up * NUP + n_dn
    k_ref[...] = key
    g_ref[...] = key >> 3


def _tc_keys(x2):
    bs = 512
    n = B * L // 128
    return pl.pallas_call(
        _idx_body,
        grid=(n // bs,),
        in_specs=[pl.BlockSpec((bs, 128), lambda i: (i, 0))],
        out_specs=[pl.BlockSpec((bs, 128), lambda i: (i, 0)),
                   pl.BlockSpec((bs, 128), lambda i: (i, 0))],
        out_shape=[jax.ShapeDtypeStruct((n, 128), jnp.int32),
                   jax.ShapeDtypeStruct((n, 128), jnp.int32)],
    )(x2)


@functools.lru_cache(maxsize=1)
def _sc_build():
    @functools.partial(
        pl.kernel, mesh=_mesh(),
        out_type=jax.ShapeDtypeStruct((TAB_ROWS, 128), jnp.float32),
        scratch_types=[
            pltpu.VMEM((M * SLAB_ROWS + 8, 128), jnp.float32),
            pltpu.VMEM((128, 128), jnp.float32),
            pltpu.SemaphoreType.DMA,
        ],
        compiler_params=_SC_PARAMS,
    )
    def k(eps_hbm, tab_hbm, slab_v, stage_v, sem):
        wid = lax.axis_index("s") * NC + lax.axis_index("c")
        lane = jnp.arange(M, dtype=jnp.int32)

        def do_al(i, _):
            al = wid * AL_PER_W + i
            a = al // L
            l = al % L
            pidx = (a * M + lane) * L + l
            p0 = pidx * PLANE
            r0a = (p0 >> 10) << 3
            r0c = jnp.minimum(r0a, EPS_ROWS - SLAB_ROWS)
            shift = p0 - (r0c << 7)
            handles = []
            for m in range(M):
                pm = ((a * M + m) * L + l) * PLANE
                r0m = jnp.minimum((pm >> 10) << 3, EPS_ROWS - SLAB_ROWS)
                handles.append(pltpu.async_copy(
                    eps_hbm.at[pl.ds(pl.multiple_of(r0m, 8), SLAB_ROWS), :],
                    slab_v.at[pl.ds(m * SLAB_ROWS, SLAB_ROWS), :],
                    sem))
            for h in handles:
                h.wait()

            base_idx = lane * (SLAB_ROWS * 128) + shift

            for ci, (q0, nrows) in enumerate(_CHUNKS):
                base2 = base_idx + q0

                def do_r(r, _):
                    q8 = r * 8
                    for j in range(8):
                        flat = base2 + (q8 + j)
                        row = plsc.load_gather(
                            slab_v, [flat >> 7, flat & 127])
                        stage_v[r, pl.ds(j * M, M)] = row
                    return 0

                lax.fori_loop(0, nrows, do_r, 0)
                pltpu.sync_copy(
                    stage_v.at[pl.ds(0, nrows), :],
                    tab_hbm.at[pl.ds(al * ROWS_PER_AL + ci * 128, nrows), :])
            return 0

        lax.fori_loop(0, AL_PER_W, do_al, 0)

    return k


@functools.lru_cache(maxsize=1)
def _sc_gather():
    scratch = [pltpu.VMEM((N_BATCH, ROWS_PER_BATCH), jnp.int32),
               pltpu.VMEM((N_BATCH, ROWS_PER_BATCH), jnp.int32)]
    scratch += [pltpu.VMEM((ROWS_PER_BATCH, 128), jnp.float32) for _ in range(NBUF)]
    scratch += [pltpu.VMEM((SAMPLES_PER_W // 128, 128), jnp.float32),
                pltpu.SemaphoreType.DMA]

    @functools.partial(
        pl.kernel, mesh=_mesh(),
        out_type=jax.ShapeDtypeStruct((B // 128, 128), jnp.float32),
        scratch_types=scratch,
        compiler_params=_SC_PARAMS,
    )
    def k(tab_hbm, g_hbm, k_hbm, out_hbm, gv, kv, r0, r1, r2, r3, res_v, sem):
        ring = (r0, r1, r2, r3)
        wid = lax.axis_index("s") * NC + lax.axis_index("c")
        pltpu.sync_copy(g_hbm.at[pl.ds(wid * N_BATCH, N_BATCH), :], gv)
        pltpu.sync_copy(k_hbm.at[pl.ds(wid * N_BATCH, N_BATCH), :], kv)
        for b in range(NBUF):
            pltpu.async_copy(tab_hbm.at[gv.at[b]], ring[b], sem)

        lane = jnp.arange(M, dtype=jnp.int32)
        zero = jnp.zeros((M,), jnp.int32)

        def sample_prod(slot, jj, half):
            base = half * L
            jjv = zero + jj

            def val(r):
                rv = zero + (base + r)
                kvec = plsc.load_gather(kv, [jjv, rv])
                off = (kvec & 7) * M + lane
                return plsc.load_gather(slot, [rv, off])

            accs = tuple(val(u) for u in range(4))

            def mbody(t, accs):
                r = t * 4
                return tuple(accs[u] * val(r + u) for u in range(4))

            a0, a1, a2, a3 = lax.fori_loop(1, L // 4, mbody, accs)
            return (a0 * a1) * (a2 * a3)

        def body(g, _):
            acc = jnp.zeros((M,), jnp.float32)
            for b8 in range(GROUP):
                jj = g * GROUP + b8
                slot = ring[b8 % NBUF]
                pltpu.make_async_copy(tab_hbm.at[gv.at[jj]], slot, sem).wait()
                pa = sample_prod(slot, jj, 0)
                pb = sample_prod(slot, jj, 1)
                acc = jnp.where(lane == 2 * b8, jnp.sum(pa), acc)
                acc = jnp.where(lane == 2 * b8 + 1, jnp.sum(pb), acc)

                @pl.when(jj + NBUF < N_BATCH)
                def _issue():
                    pltpu.async_copy(tab_hbm.at[gv.at[jj + NBUF]], slot, sem)
            s16 = g * M
            res_v[s16 >> 7, pl.ds(s16 & 127, M)] = acc
            return 0

        lax.fori_loop(0, N_BATCH // GROUP, body, 0)
        n_out = SAMPLES_PER_W // 128
        pltpu.sync_copy(res_v, out_hbm.at[pl.ds(wid * n_out, n_out), :])

    return k


def kernel(inputs, epsilon):
    x2 = inputs.reshape(B * L // 128, 128)
    g_arr, k_arr = _tc_keys(x2)
    eps2 = epsilon.reshape(EPS_ROWS, 128)
    table = _sc_build()(eps2)
    out = _sc_gather()(table, g_arr, k_arr)
    return out.reshape(B)


# speedup vs baseline: 1.2308x; 1.2308x over previous
import functools

import jax
import jax.numpy as jnp
from jax import lax
from jax.experimental import pallas as pl
from jax.experimental.pallas import tpu as pltpu
from jax.experimental.pallas import tpu_sc as plsc

B = 8192
L = 64
M = 16
NUP = 65
A = 4
PLANE = NUP * NUP
PLANE_P = 4288
ROWS_PER_AL = PLANE_P * M // 128
TAB_ROWS = A * L * ROWS_PER_AL
EPS_ROWS = A * M * L * PLANE // 128

NC = 2
NS = 16
NW = NC * NS
SAMPLES_PER_W = B // NW
ROWS_PER_BATCH = 128
N_BATCH = SAMPLES_PER_W * L // ROWS_PER_BATCH
NBUF = 4
GROUP = 8

AL_PER_W = (A * L) // NW
SLAB_ROWS = 48

_CHUNKS = ((0, 128), (1024, 128), (2048, 128), (3072, 128), (4096, 24))

_SC_PARAMS = pltpu.CompilerParams(
    needs_layout_passes=False, use_tc_tiling_on_sc=True)


def _mesh():
    return plsc.VectorSubcoreMesh(
        core_axis_name="c", subcore_axis_name="s", num_cores=NC, num_subcores=NS)


def _idx_body(x_ref, g_ref, k_ref):
    x = x_ref[...]
    up = (x & 1).astype(jnp.float32)
    dn = ((x >> 1) & 1).astype(jnp.float32)
    i = lax.broadcasted_iota(jnp.int32, (128, 128), 0)
    j = lax.broadcasted_iota(jnp.int32, (128, 128), 1)
    tri = ((i < j) & ((i // L) == (j // L))).astype(jnp.float32)
    n_up = jnp.dot(up, tri, preferred_element_type=jnp.float32).astype(jnp.int32)
    n_dn = jnp.dot(dn, tri, preferred_element_type=jnp.float32).astype(jnp.int32)
    site = lax.broadcasted_iota(jnp.int32, x.shape, 1) & (L - 1)
    key = (x * L + site) * PLANE_P + n_up * NUP + n_dn
    k_ref[...] = key
    g_ref[...] = key >> 3


def _tc_keys(x2):
    bs = 512
    n = B * L // 128
    return pl.pallas_call(
        _idx_body,
        grid=(n // bs,),
        in_specs=[pl.BlockSpec((bs, 128), lambda i: (i, 0))],
        out_specs=[pl.BlockSpec((bs, 128), lambda i: (i, 0)),
                   pl.BlockSpec((bs, 128), lambda i: (i, 0))],
        out_shape=[jax.ShapeDtypeStruct((n, 128), jnp.int32),
                   jax.ShapeDtypeStruct((n, 128), jnp.int32)],
    )(x2)


@functools.lru_cache(maxsize=1)
def _sc_build():
    @functools.partial(
        pl.kernel, mesh=_mesh(),
        out_type=jax.ShapeDtypeStruct((TAB_ROWS, 128), jnp.float32),
        scratch_types=[
            pltpu.VMEM((M * SLAB_ROWS + 8, 128), jnp.float32),
            pltpu.VMEM((128, 128), jnp.float32),
            pltpu.SemaphoreType.DMA,
        ],
        compiler_params=_SC_PARAMS,
    )
    def k(eps_hbm, tab_hbm, slab_v, stage_v, sem):
        wid = lax.axis_index("s") * NC + lax.axis_index("c")
        lane = jnp.arange(M, dtype=jnp.int32)

        def do_al(i, _):
            al = wid * AL_PER_W + i
            a = al // L
            l = al % L
            pidx = (a * M + lane) * L + l
            p0 = pidx * PLANE
            r0a = (p0 >> 10) << 3
            r0c = jnp.minimum(r0a, EPS_ROWS - SLAB_ROWS)
            shift = p0 - (r0c << 7)
            handles = []
            for m in range(M):
                pm = ((a * M + m) * L + l) * PLANE
                r0m = jnp.minimum((pm >> 10) << 3, EPS_ROWS - SLAB_ROWS)
                handles.append(pltpu.async_copy(
                    eps_hbm.at[pl.ds(pl.multiple_of(r0m, 8), SLAB_ROWS), :],
                    slab_v.at[pl.ds(m * SLAB_ROWS, SLAB_ROWS), :],
                    sem))
            for h in handles:
                h.wait()

            base_idx = lane * (SLAB_ROWS * 128) + shift

            for ci, (q0, nrows) in enumerate(_CHUNKS):
                base2 = base_idx + q0

                def do_r(r2, _):
                    for rr in range(2):
                        r = r2 * 2 + rr
                        q8 = r * 8
                        rows = []
                        for j in range(8):
                            flat = base2 + (q8 + j)
                            rows.append(plsc.load_gather(
                                slab_v, [flat >> 7, flat & 127]))
                        for j in range(8):
                            stage_v[r, pl.ds(j * M, M)] = rows[j]
                    return 0

                lax.fori_loop(0, nrows // 2, do_r, 0)
                pltpu.sync_copy(
                    stage_v.at[pl.ds(0, nrows), :],
                    tab_hbm.at[pl.ds(al * ROWS_PER_AL + ci * 128, nrows), :])
            return 0

        lax.fori_loop(0, AL_PER_W, do_al, 0)

    return k


@functools.lru_cache(maxsize=1)
def _sc_gather():
    scratch = [pltpu.VMEM((N_BATCH, ROWS_PER_BATCH), jnp.int32),
               pltpu.VMEM((N_BATCH, ROWS_PER_BATCH), jnp.int32)]
    scratch += [pltpu.VMEM((ROWS_PER_BATCH, 128), jnp.float32) for _ in range(NBUF)]
    scratch += [pltpu.VMEM((SAMPLES_PER_W // 128, 128), jnp.float32),
                pltpu.SemaphoreType.DMA]

    @functools.partial(
        pl.kernel, mesh=_mesh(),
        out_type=jax.ShapeDtypeStruct((B // 128, 128), jnp.float32),
        scratch_types=scratch,
        compiler_params=_SC_PARAMS,
    )
    def k(tab_hbm, g_hbm, k_hbm, out_hbm, gv, kv, r0, r1, r2, r3, res_v, sem):
        ring = (r0, r1, r2, r3)
        wid = lax.axis_index("s") * NC + lax.axis_index("c")
        pltpu.sync_copy(g_hbm.at[pl.ds(wid * N_BATCH, N_BATCH), :], gv)
        pltpu.sync_copy(k_hbm.at[pl.ds(wid * N_BATCH, N_BATCH), :], kv)
        for b in range(NBUF):
            pltpu.async_copy(tab_hbm.at[gv.at[b]], ring[b], sem)

        lane = jnp.arange(M, dtype=jnp.int32)
        zero = jnp.zeros((M,), jnp.int32)

        def sample_prod(slot, jj, half):
            base = half * L
            jjv = zero + jj

            def val(r):
                rv = zero + (base + r)
                kvec = plsc.load_gather(kv, [jjv, rv])
                off = (kvec & 7) * M + lane
                return plsc.load_gather(slot, [rv, off])

            accs = tuple(val(u) for u in range(4))

            def mbody(t, accs):
                r = t * 4
                return tuple(accs[u] * val(r + u) for u in range(4))

            a0, a1, a2, a3 = lax.fori_loop(1, L // 4, mbody, accs)
            return (a0 * a1) * (a2 * a3)

        def body(g, _):
            acc = jnp.zeros((M,), jnp.float32)
            for b8 in range(GROUP):
                jj = g * GROUP + b8
                slot = ring[b8 % NBUF]
                pltpu.make_async_copy(tab_hbm.at[gv.at[jj]], slot, sem).wait()
                pa = sample_prod(slot, jj, 0)
                pb = sample_prod(slot, jj, 1)
                acc = jnp.where(lane == 2 * b8, jnp.sum(pa), acc)
                acc = jnp.where(lane == 2 * b8 + 1, jnp.sum(pb), acc)

                @pl.when(jj + NBUF < N_BATCH)
                def _issue():
                    pltpu.async_copy(tab_hbm.at[gv.at[jj + NBUF]], slot, sem)
            s16 = g * M
            res_v[s16 >> 7, pl.ds(s16 & 127, M)] = acc
            return 0

        lax.fori_loop(0, N_BATCH // GROUP, body, 0)
        n_out = SAMPLES_PER_W // 128
        pltpu.sync_copy(res_v, out_hbm.at[pl.ds(wid * n_out, n_out), :])

    return k


def kernel(inputs, epsilon):
    x2 = inputs.reshape(B * L // 128, 128)
    g_arr, k_arr = _tc_keys(x2)
    eps2 = epsilon.reshape(EPS_ROWS, 128)
    table = _sc_build()(eps2)
    out = _sc_gather()(table, g_arr, k_arr)
    return out.reshape(B)


# speedup vs baseline: 1.2413x; 1.0085x over previous
import functools

import jax
import jax.numpy as jnp
from jax import lax
from jax.experimental import pallas as pl
from jax.experimental.pallas import tpu as pltpu
from jax.experimental.pallas import tpu_sc as plsc

B = 8192
L = 64
M = 16
NUP = 65
A = 4
PLANE = NUP * NUP
PLANE_P = 4288
ROWS_PER_AL = PLANE_P * M // 128
TAB_ROWS = A * L * ROWS_PER_AL
EPS_ROWS = A * M * L * PLANE // 128

NC = 2
NS = 16
NW = NC * NS
SAMPLES_PER_W = B // NW
ROWS_PER_BATCH = 128
N_BATCH = SAMPLES_PER_W * L // ROWS_PER_BATCH
NBUF = 4
GROUP = 8

AL_PER_W = (A * L) // NW
SLAB_ROWS = 48

_CHUNKS = ((0, 128), (1024, 128), (2048, 128), (3072, 128), (4096, 24))

_SC_PARAMS = pltpu.CompilerParams(
    needs_layout_passes=False, use_tc_tiling_on_sc=True)


def _mesh():
    return plsc.VectorSubcoreMesh(
        core_axis_name="c", subcore_axis_name="s", num_cores=NC, num_subcores=NS)


def _idx_body(x_ref, g_ref, k_ref):
    x = x_ref[...]
    up = (x & 1).astype(jnp.float32)
    dn = ((x >> 1) & 1).astype(jnp.float32)
    i = lax.broadcasted_iota(jnp.int32, (128, 128), 0)
    j = lax.broadcasted_iota(jnp.int32, (128, 128), 1)
    tri = ((i < j) & ((i // L) == (j // L))).astype(jnp.float32)
    n_up = jnp.dot(up, tri, preferred_element_type=jnp.float32).astype(jnp.int32)
    n_dn = jnp.dot(dn, tri, preferred_element_type=jnp.float32).astype(jnp.int32)
    site = lax.broadcasted_iota(jnp.int32, x.shape, 1) & (L - 1)
    key = (x * L + site) * PLANE_P + n_up * NUP + n_dn
    k_ref[...] = key
    g_ref[...] = key >> 3


def _tc_keys(x2):
    bs = 512
    n = B * L // 128
    return pl.pallas_call(
        _idx_body,
        grid=(n // bs,),
        in_specs=[pl.BlockSpec((bs, 128), lambda i: (i, 0))],
        out_specs=[pl.BlockSpec((bs, 128), lambda i: (i, 0)),
                   pl.BlockSpec((bs, 128), lambda i: (i, 0))],
        out_shape=[jax.ShapeDtypeStruct((n, 128), jnp.int32),
                   jax.ShapeDtypeStruct((n, 128), jnp.int32)],
    )(x2)


@functools.lru_cache(maxsize=1)
def _sc_build():
    @functools.partial(
        pl.kernel, mesh=_mesh(),
        out_type=jax.ShapeDtypeStruct((TAB_ROWS, 128), jnp.float32),
        scratch_types=[
            pltpu.VMEM((M * SLAB_ROWS + 8, 128), jnp.float32),
            pltpu.VMEM((128, 128), jnp.float32),
            pltpu.SemaphoreType.DMA,
        ],
        compiler_params=_SC_PARAMS,
    )
    def k(eps_hbm, tab_hbm, slab_v, stage_v, sem):
        wid = lax.axis_index("s") * NC + lax.axis_index("c")
        lane = jnp.arange(M, dtype=jnp.int32)

        def do_al(i, _):
            al = wid * AL_PER_W + i
            a = al // L
            l = al % L
            pidx = (a * M + lane) * L + l
            p0 = pidx * PLANE
            r0a = (p0 >> 10) << 3
            r0c = jnp.minimum(r0a, EPS_ROWS - SLAB_ROWS)
            shift = p0 - (r0c << 7)
            handles = []
            for m in range(M):
                pm = ((a * M + m) * L + l) * PLANE
                r0m = jnp.minimum((pm >> 10) << 3, EPS_ROWS - SLAB_ROWS)
                handles.append(pltpu.async_copy(
                    eps_hbm.at[pl.ds(pl.multiple_of(r0m, 8), SLAB_ROWS), :],
                    slab_v.at[pl.ds(m * SLAB_ROWS, SLAB_ROWS), :],
                    sem))
            for h in handles:
                h.wait()

            base_idx = lane * (SLAB_ROWS * 128) + shift

            for ci, (q0, nrows) in enumerate(_CHUNKS):
                base2 = base_idx + q0

                def do_r(r4, _):
                    rows = []
                    for rr in range(4):
                        r = r4 * 4 + rr
                        for j in range(8):
                            flat = base2 + (r * 8 + j)
                            rows.append(plsc.load_gather(
                                slab_v, [flat >> 7, flat & 127]))
                    for rr in range(4):
                        r = r4 * 4 + rr
                        for j in range(8):
                            stage_v[r, pl.ds(j * M, M)] = rows[rr * 8 + j]
                    return 0

                lax.fori_loop(0, nrows // 4, do_r, 0)
                pltpu.sync_copy(
                    stage_v.at[pl.ds(0, nrows), :],
                    tab_hbm.at[pl.ds(al * ROWS_PER_AL + ci * 128, nrows), :])
            return 0

        lax.fori_loop(0, AL_PER_W, do_al, 0)

    return k


@functools.lru_cache(maxsize=1)
def _sc_gather():
    scratch = [pltpu.VMEM((N_BATCH, ROWS_PER_BATCH), jnp.int32),
               pltpu.VMEM((N_BATCH, ROWS_PER_BATCH), jnp.int32)]
    scratch += [pltpu.VMEM((ROWS_PER_BATCH, 128), jnp.float32) for _ in range(NBUF)]
    scratch += [pltpu.VMEM((SAMPLES_PER_W // 128, 128), jnp.float32),
                pltpu.SemaphoreType.DMA]

    @functools.partial(
        pl.kernel, mesh=_mesh(),
        out_type=jax.ShapeDtypeStruct((B // 128, 128), jnp.float32),
        scratch_types=scratch,
        compiler_params=_SC_PARAMS,
    )
    def k(tab_hbm, g_hbm, k_hbm, out_hbm, gv, kv, r0, r1, r2, r3, res_v, sem):
        ring = (r0, r1, r2, r3)
        wid = lax.axis_index("s") * NC + lax.axis_index("c")
        pltpu.sync_copy(g_hbm.at[pl.ds(wid * N_BATCH, N_BATCH), :], gv)
        pltpu.sync_copy(k_hbm.at[pl.ds(wid * N_BATCH, N_BATCH), :], kv)
        for b in range(NBUF):
            pltpu.async_copy(tab_hbm.at[gv.at[b]], ring[b], sem)

        lane = jnp.arange(M, dtype=jnp.int32)
        zero = jnp.zeros((M,), jnp.int32)

        def sample_prod(slot, jj, half):
            base = half * L
            jjv = zero + jj

            def val(r):
                rv = zero + (base + r)
                kvec = plsc.load_gather(kv, [jjv, rv])
                off = (kvec & 7) * M + lane
                return plsc.load_gather(slot, [rv, off])

            accs = tuple(val(u) for u in range(4))

            def mbody(t, accs):
                r = t * 4
                return tuple(accs[u] * val(r + u) for u in range(4))

            a0, a1, a2, a3 = lax.fori_loop(1, L // 4, mbody, accs)
            return (a0 * a1) * (a2 * a3)

        def body(g, _):
            acc = jnp.zeros((M,), jnp.float32)
            for b8 in range(GROUP):
                jj = g * GROUP + b8
                slot = ring[b8 % NBUF]
                pltpu.make_async_copy(tab_hbm.at[gv.at[jj]], slot, sem).wait()
                pa = sample_prod(slot, jj, 0)
                pb = sample_prod(slot, jj, 1)
                acc = jnp.where(lane == 2 * b8, jnp.sum(pa), acc)
                acc = jnp.where(lane == 2 * b8 + 1, jnp.sum(pb), acc)

                @pl.when(jj + NBUF < N_BATCH)
                def _issue():
                    pltpu.async_copy(tab_hbm.at[gv.at[jj + NBUF]], slot, sem)
            s16 = g * M
            res_v[s16 >> 7, pl.ds(s16 & 127, M)] = acc
            return 0

        lax.fori_loop(0, N_BATCH // GROUP, body, 0)
        n_out = SAMPLES_PER_W // 128
        pltpu.sync_copy(res_v, out_hbm.at[pl.ds(wid * n_out, n_out), :])

    return k


def kernel(inputs, epsilon):
    x2 = inputs.reshape(B * L // 128, 128)
    g_arr, k_arr = _tc_keys(x2)
    eps2 = epsilon.reshape(EPS_ROWS, 128)
    table = _sc_build()(eps2)
    out = _sc_gather()(table, g_arr, k_arr)
    return out.reshape(B)
